# baseline (speedup 1.0000x reference)
import functools

import jax
import jax.numpy as jnp
from jax import lax
from jax.experimental import pallas as pl
from jax.experimental.pallas import tpu as pltpu
from jax.experimental.pallas import tpu_sc as plsc

BATCH = 1024
SEQ = 200
EMB = 64
LANES = 128
NC = 2
NS = 16
NW = NC * NS

TOTAL = BATCH * SEQ
ROWS_PER_W = TOTAL // NW
CHUNK = 256
NCHUNK = ROWS_PER_W // CHUNK
VCHUNK = CHUNK // 2
IROWS_PER_W = ROWS_PER_W // LANES
IROWS_STRIDE = 56
IROWS_PER_CHUNK = CHUNK // LANES
GROUP = 8

_mesh = plsc.VectorSubcoreMesh(
    core_axis_name="c", subcore_axis_name="s", num_cores=NC, num_subcores=NS
)


@functools.partial(
    pl.kernel,
    out_type=jax.ShapeDtypeStruct((TOTAL // 2, LANES), jnp.float32),
    mesh=_mesh,
    scratch_types=[
        pltpu.VMEM((IROWS_STRIDE, LANES), jnp.int32),
        pltpu.VMEM((IROWS_STRIDE * LANES,), jnp.int32),
        pltpu.VMEM((CHUNK, LANES), jnp.float32),
        pltpu.VMEM((CHUNK, LANES), jnp.float32),
        pltpu.VMEM((VCHUNK, LANES), jnp.float32),
        pltpu.VMEM((VCHUNK, LANES), jnp.float32),
        pltpu.VMEM((SEQ // 2, LANES), jnp.float32),
        pltpu.SemaphoreType.DMA,
        pltpu.SemaphoreType.DMA,
        pltpu.SemaphoreType.DMA,
        pltpu.SemaphoreType.DMA,
    ],
)
def _embed_sc(idx_hbm, half_hbm, tok_hbm, pos_hbm, out_hbm,
              idx_v, half_flat, gath_a, gath_b, pack_a, pack_b, pos_v,
              gsem_a, gsem_b, wsem_a, wsem_b):
    wid = lax.axis_index("s") * NC + lax.axis_index("c")
    irow0 = wid * IROWS_STRIDE
    vrow0 = wid * (ROWS_PER_W // 2)

    pltpu.sync_copy(idx_hbm.at[pl.ds(irow0, IROWS_STRIDE)], idx_v)
    pltpu.sync_copy(
        half_hbm.at[pl.ds(irow0 * LANES, IROWS_STRIDE * LANES)], half_flat)
    pltpu.sync_copy(pos_hbm, pos_v)

    gath = (gath_a, gath_b)
    pack = (pack_a, pack_b)
    gsem = (gsem_a, gsem_b)
    wsem = (wsem_a, wsem_b)

    def start_gather(c):
        p = c % 2
        gs = []
        for h in range(IROWS_PER_CHUNK):
            gs.append(pltpu.async_copy(
                tok_hbm.at[idx_v.at[IROWS_PER_CHUNK * c + h]],
                gath[p].at[pl.ds(h * LANES, LANES)],
                gsem[p],
            ))
        return gs

    def compute(c):
        p = c % 2
        g, o = gath[p], pack[p]
        s0 = (CHUNK * c) % SEQ

        def body(t, s):
            hvec = half_flat[pl.ds(c * CHUNK + GROUP * t, 16)]
            for r in range(GROUP):
                j = GROUP * t + r
                v = (GROUP // 2) * t + r // 2
                e = r % 2
                hoff = hvec[r]
                se = s + r
                se = jnp.where(se >= SEQ, se - SEQ, se)
                poff = (se % 2) * EMB
                for k in range(EMB // 16):
                    o[v, pl.ds(e * EMB + k * 16, 16)] = (
                        g[j, pl.ds(hoff + k * 16, 16)]
                        + pos_v[se // 2, pl.ds(poff + k * 16, 16)]
                    )
            s = s + GROUP
            return jnp.where(s >= SEQ, s - SEQ, s)

        lax.fori_loop(0, CHUNK // GROUP, body, jnp.int32(s0))

    pending_g = [None, None]
    pending_w = [None, None]

    for c in range(NCHUNK + 1):
        if c < NCHUNK:
            p = c % 2
            if pending_w[p] is not None:
                pending_w[p].wait()
                pending_w[p] = None
            pending_g[p] = start_gather(c)
        if c >= 1:
            q = (c - 1) % 2
            for g in pending_g[q]:
                g.wait()
            pending_g[q] = None
            compute(c - 1)
            pending_w[q] = pltpu.async_copy(
                pack[q],
                out_hbm.at[pl.ds(vrow0 + (c - 1) * VCHUNK, VCHUNK)],
                wsem[q],
            )

    for p in range(2):
        if pending_w[p] is not None:
            pending_w[p].wait()


def _worker_blocks(x):
    x = x.reshape(NW, IROWS_PER_W, LANES)
    x = jnp.pad(x, ((0, 0), (0, IROWS_STRIDE - IROWS_PER_W), (0, 0)))
    return x.reshape(NW * IROWS_STRIDE, LANES)


def kernel(inputs, token_table, position_table):
    flat = inputs.reshape(-1).astype(jnp.int32)
    idx = _worker_blocks(flat // 2)
    half = _worker_blocks((flat % 2) * EMB).reshape(-1)
    tok2 = token_table.reshape(token_table.shape[0] // 2, 2 * EMB)
    pos2 = position_table.reshape(SEQ // 2, 2 * EMB)
    out = _embed_sc(idx, half, tok2, pos2)
    return out.reshape(BATCH, SEQ, EMB)
